# baseline (speedup 1.0000x reference)
import functools

import jax
import jax.numpy as jnp
from jax.experimental import pallas as pl
from jax.experimental.pallas import tpu as pltpu

CHUNK = 64
D_CONV = 4
NCORES = 1
HP = 64
HIGH = jax.lax.Precision.HIGHEST


def _proj_kernel(x_ref, w_ref, o_ref):
    xb = x_ref[...].astype(jnp.bfloat16)
    o_ref[...] = jax.lax.dot_general(
        xb, w_ref[...],
        dimension_numbers=(((1,), (0,)), ((), ())),
        preferred_element_type=jnp.float32)


def _proj_xbc(x0, w1t_bf):
    s, dm = x0.shape
    n = w1t_bf.shape[1]
    bm, bn = 512, 1024
    mh = s // bm // NCORES
    return pl.pallas_call(
        _proj_kernel,
        grid=(NCORES, mh, n // bn),
        in_specs=[
            pl.BlockSpec((bm, dm), lambda c, i, j: (c * mh + i, 0)),
            pl.BlockSpec((dm, bn), lambda c, i, j: (0, j)),
        ],
        out_specs=pl.BlockSpec((bm, bn), lambda c, i, j: (c * mh + i, j)),
        out_shape=jax.ShapeDtypeStruct((s, n), jnp.float32),
        compiler_params=pltpu.CompilerParams(
            dimension_semantics=("core_parallel", "parallel", "parallel")),
        name="proj_xbc",
    )(x0, w1t_bf)


def _ssd_kernel(nheads,
                cur_ref, prev_ref, xb2_ref, wdt_ref, wdtp_ref,
                cw_ref, cb_ref, ap_ref, dtb_ref, app_ref, dtbp_ref,
                y_ref, inv_ref, state_ref, carry_ref):
    i = pl.program_id(1)
    f32 = jnp.float32
    bf = jnp.bfloat16
    nh_loc = nheads // NCORES
    part = nh_loc * HP
    srows = cur_ref.shape[0]

    @pl.when(i == 0)
    def _init():
        state_ref[...] = jnp.zeros_like(state_ref)
        carry_ref[...] = jnp.zeros_like(carry_ref)

    cur = cur_ref[...]
    tail8 = jnp.where(i > 0, prev_ref[...], 0.0)
    ext = jnp.concatenate([cur, tail8], axis=0).astype(bf)
    ie = jax.lax.broadcasted_iota(jnp.int32, (srows, srows + 8), 0)
    je = jax.lax.broadcasted_iota(jnp.int32, (srows, srows + 8), 1)
    conv = cur * cw_ref[3:4, :] + cb_ref[...]
    for k in (1, 2, 3):
        mk = (((je == ie - k) & (je < srows)) |
              ((je == srows + 8 - k + ie) & (ie < k)))
        shifted = jax.lax.dot_general(
            mk.astype(bf), ext, dimension_numbers=(((1,), (0,)), ((), ())),
            preferred_element_type=f32)
        conv += shifted * cw_ref[3 - k:4 - k, :]

    nbl = xb2_ref.shape[0]
    rows = nbl * srows
    xall = xb2_ref[...].reshape(rows, xb2_ref.shape[2]).astype(bf)
    dtraw = jax.lax.dot_general(
        xall, wdt_ref[...], dimension_numbers=(((1,), (0,)), ((), ())),
        preferred_element_type=f32) + dtb_ref[...]
    dt = jnp.maximum(dtraw, 0.0) + jnp.log1p(jnp.exp(-jnp.abs(dtraw)))
    a_all = ap_ref[...] * dt

    ii = jax.lax.broadcasted_iota(jnp.int32, (rows, rows), 0)
    jj = jax.lax.broadcasted_iota(jnp.int32, (rows, rows), 1)
    blkmask = ((jj <= ii) & ((ii // srows) == (jj // srows))).astype(f32)
    ahi = a_all.astype(bf)
    alo = (a_all - ahi.astype(f32)).astype(bf)
    blk_bf = blkmask.astype(bf)
    cuml = (jax.lax.dot_general(
                blk_bf, ahi, dimension_numbers=(((1,), (0,)), ((), ())),
                preferred_element_type=f32) +
            jax.lax.dot_general(
                blk_bf, alo, dimension_numbers=(((1,), (0,)), ((), ())),
                preferred_element_type=f32))
    coff = carry_ref[0:nbl, :]
    rsel = ((ii[:, 0:nbl] // srows) ==
            jax.lax.broadcasted_iota(jnp.int32, (rows, nbl), 1)).astype(f32)
    cuma = cuml + jax.lax.dot_general(
        rsel, coff, dimension_numbers=(((1,), (0,)), ((), ())),
        preferred_element_type=f32, precision=HIGH)
    en = jnp.exp(-cuma)
    inner = jax.lax.dot_general(
        blk_bf, en.astype(bf),
        dimension_numbers=(((1,), (0,)), ((), ())),
        preferred_element_type=f32)
    inner += jax.lax.dot_general(
        rsel, carry_ref[4:4 + nbl, :],
        dimension_numbers=(((1,), (0,)), ((), ())),
        preferred_element_type=f32, precision=HIGH)
    inv_ref[...] = (1.0 / (jnp.exp(cuma) * inner)).reshape(nbl, srows, nheads)
    newoff = jnp.concatenate(
        [cuma[b * srows + srows - 1:b * srows + srows, :] for b in range(nbl)],
        axis=0)
    segsum = jnp.concatenate(
        [jnp.sum(en[b * srows:(b + 1) * srows, :], axis=0, keepdims=True)
         for b in range(nbl)], axis=0)
    carry_ref[0:nbl, :] = newoff
    carry_ref[4:4 + nbl, :] = carry_ref[4:4 + nbl, :] + segsum

    nsub = srows // CHUNK
    x0 = xall[0:srows, :]
    dtraw0 = jax.lax.dot_general(
        x0, wdtp_ref[...], dimension_numbers=(((1,), (0,)), ((), ())),
        preferred_element_type=f32) + dtbp_ref[...]
    dt0 = jnp.maximum(dtraw0, 0.0) + jnp.log1p(jnp.exp(-jnp.abs(dtraw0)))
    a0 = (app_ref[...] * dt0).astype(bf)
    si = jax.lax.broadcasted_iota(jnp.int32, (srows, srows), 0)
    sj = jax.lax.broadcasted_iota(jnp.int32, (srows, srows), 1)
    btri = (sj <= si) & ((si // CHUNK) == (sj // CHUNK))
    cum0 = jax.lax.dot_general(
        btri.astype(bf), a0, dimension_numbers=(((1,), (0,)), ((), ())),
        preferred_element_type=f32)
    u = jnp.exp(cum0)
    v = jnp.exp(-cum0)
    li = jax.lax.broadcasted_iota(jnp.int32, (CHUNK, CHUNK), 0)
    lj = jax.lax.broadcasted_iota(jnp.int32, (CHUNK, CHUNK), 1)
    ltri = lj <= li
    for h in range(nh_loc):
        sl = slice(h * HP, (h + 1) * HP)
        sh = state_ref[sl, :]
        ys = []
        for sck in range(nsub):
            rs = slice(sck * CHUNK, (sck + 1) * CHUNK)
            ucol = u[rs, h:h + 1]
            vcol = v[rs, h:h + 1]
            ct = (conv[rs, sl] * ucol).astype(bf)
            bv = (conv[rs, part + h * HP:part + (h + 1) * HP]
                  * vcol).astype(bf)
            xh = conv[rs, 2 * part + h * HP:2 * part + (h + 1) * HP
                      ].astype(bf)
            g = jax.lax.dot_general(
                ct, bv, dimension_numbers=(((1,), (1,)), ((), ())),
                preferred_element_type=f32)
            gm = jnp.where(ltri, g, 0.0).astype(bf)
            ys.append(jax.lax.dot_general(
                jnp.concatenate([gm, ct], axis=1),
                jnp.concatenate([xh, sh.astype(bf)], axis=0),
                dimension_numbers=(((1,), (0,)), ((), ())),
                preferred_element_type=f32))
            sc = jax.lax.dot_general(
                bv, xh, dimension_numbers=(((0,), (0,)), ((), ())),
                preferred_element_type=f32)
            sh = (sh + sc) * u[sck * CHUNK + CHUNK - 1:
                               sck * CHUNK + CHUNK, h:h + 1]
        state_ref[sl, :] = sh
        y_ref[:, sl] = ys[0] if nsub == 1 else jnp.concatenate(ys, axis=0)


def _ssd(proj0p, x, wdt_bf, wdtp_bf, cwp, cbp, ap_row, dtb_row,
         app_row, dtbp_row):
    nb, s, dm = x.shape
    dcc = proj0p.shape[1]
    nheads = ap_row.shape[1]
    slab = dcc // NCORES
    nbl = nb // NCORES
    srows = 2 * CHUNK
    nsteps = s // srows
    kfn = functools.partial(_ssd_kernel, nheads)
    return pl.pallas_call(
        kfn,
        grid=(NCORES, nsteps),
        in_specs=[
            pl.BlockSpec((srows, slab), lambda c, i: (i, c)),
            pl.BlockSpec((8, slab),
                         lambda c, i: (jnp.maximum(srows // 8 * i - 1, 0),
                                       c * (slab // slab))),
            pl.BlockSpec((nbl, srows, dm), lambda c, i: (c, i, 0)),
            pl.BlockSpec((dm, nheads), lambda c, i: (0, 0)),
            pl.BlockSpec((dm, 128), lambda c, i: (0, c)),
            pl.BlockSpec((D_CONV, slab), lambda c, i: (0, c)),
            pl.BlockSpec((1, slab), lambda c, i: (0, c)),
            pl.BlockSpec((1, nheads), lambda c, i: (0, 0)),
            pl.BlockSpec((1, nheads), lambda c, i: (0, 0)),
            pl.BlockSpec((1, 128), lambda c, i: (0, c)),
            pl.BlockSpec((1, 128), lambda c, i: (0, c)),
        ],
        out_specs=[
            pl.BlockSpec((srows, slab // 3), lambda c, i: (i, c)),
            pl.BlockSpec((nbl, srows, nheads), lambda c, i: (c, i, 0)),
        ],
        out_shape=[
            jax.ShapeDtypeStruct((s, dcc // 3), jnp.float32),
            jax.ShapeDtypeStruct((nb, s, nheads), jnp.float32),
        ],
        scratch_shapes=[
            pltpu.VMEM((slab // 3, HP), jnp.float32),
            pltpu.VMEM((8, nheads), jnp.float32),
        ],
        compiler_params=pltpu.CompilerParams(
            dimension_semantics=("core_parallel", "arbitrary")),
        name="conv_ssd_norm",
    )(proj0p, proj0p, x, wdt_bf, wdtp_bf, cwp, cbp, ap_row, dtb_row,
      app_row, dtbp_row)


def _out_kernel(nheads, y_ref, inv_ref, w_ref, o_ref):
    f32 = jnp.float32
    bm = y_ref.shape[0]
    di = y_ref.shape[1]
    hp = di // nheads
    inv = inv_ref[...].reshape(bm, nheads)
    invx = jnp.concatenate(
        [jnp.broadcast_to(inv[:, h:h + 1], (bm, hp)) for h in range(nheads)],
        axis=1)
    z = (y_ref[...] * invx).astype(jnp.bfloat16)
    o = jax.lax.dot_general(
        z, w_ref[...], dimension_numbers=(((1,), (0,)), ((), ())),
        preferred_element_type=f32)
    o_ref[...] = o.reshape(1, bm, o.shape[1])


def _out_proj(y0, invn, wot_bf):
    nb, s, nheads = invn.shape
    di = y0.shape[1]
    dm = wot_bf.shape[1]
    bm = 1024
    nbl = nb // NCORES
    kfn = functools.partial(_out_kernel, nheads)
    return pl.pallas_call(
        kfn,
        grid=(NCORES, s // bm, nbl),
        in_specs=[
            pl.BlockSpec((bm, di), lambda c, m, b: (m, 0)),
            pl.BlockSpec((1, bm, nheads),
                         lambda c, m, b: (c * nbl + b, m, 0)),
            pl.BlockSpec((di, dm), lambda c, m, b: (0, 0)),
        ],
        out_specs=pl.BlockSpec((1, bm, dm),
                               lambda c, m, b: (c * nbl + b, m, 0)),
        out_shape=jax.ShapeDtypeStruct((nb, s, dm), jnp.float32),
        compiler_params=pltpu.CompilerParams(
            dimension_semantics=("core_parallel", "parallel", "parallel")),
        name="scale_outproj",
    )(y0, invn, wot_bf)


def kernel(x, W_in, conv_w, conv_b, A_param, dt_bias, W_out):
    nb, s, dm = x.shape
    nheads = A_param.shape[0]
    dcc = conv_w.shape[0]
    nh_loc = nheads // NCORES

    def permute_cols(a):
        lead = a.shape[:-1]
        ap = a.reshape(*lead, 3, NCORES, nh_loc, HP)
        ap = jnp.moveaxis(ap, -4, -3)
        return ap.reshape(*lead, dcc)

    x0 = x[0]
    w1t_bf = permute_cols(W_in[:dcc].T).astype(jnp.bfloat16)
    wdt = W_in[dcc:].T
    wdt_bf = wdt.astype(jnp.bfloat16)
    wdtp = jnp.zeros((dm, NCORES * 128), jnp.float32)
    app_row = jnp.zeros((1, NCORES * 128), jnp.float32)
    dtbp_row = jnp.zeros((1, NCORES * 128), jnp.float32)
    for c in range(NCORES):
        hs = slice(c * nh_loc, (c + 1) * nh_loc)
        cs = slice(c * 128, c * 128 + nh_loc)
        wdtp = wdtp.at[:, cs].set(wdt[:, hs])
        app_row = app_row.at[0, cs].set(A_param[hs])
        dtbp_row = dtbp_row.at[0, cs].set(dt_bias[hs])
    wdtp_bf = wdtp.astype(jnp.bfloat16)
    cwp = permute_cols(conv_w.T)
    cbp = permute_cols(conv_b.reshape(1, dcc))
    ap_row = A_param.reshape(1, nheads)
    dtb_row = dt_bias.reshape(1, nheads)
    wot_bf = W_out.T.astype(jnp.bfloat16)

    proj0p = _proj_xbc(x0, w1t_bf)
    y0p, invn = _ssd(proj0p, x, wdt_bf, wdtp_bf, cwp, cbp, ap_row, dtb_row,
                     app_row, dtbp_row)
    return _out_proj(y0p, invn, wot_bf)


# speedup vs baseline: 1.0940x; 1.0940x over previous
import functools

import jax
import jax.numpy as jnp
from jax.experimental import pallas as pl
from jax.experimental.pallas import tpu as pltpu

CHUNK = 64
D_CONV = 4
NCORES = 1
HP = 64
HIGH = jax.lax.Precision.HIGHEST


def _proj_kernel(x_ref, w_ref, o_ref):
    xb = x_ref[...].astype(jnp.bfloat16)
    o_ref[...] = jax.lax.dot_general(
        xb, w_ref[...],
        dimension_numbers=(((1,), (0,)), ((), ())),
        preferred_element_type=jnp.float32)


def _proj_xbc(x0, w1t_bf):
    s, dm = x0.shape
    n = w1t_bf.shape[1]
    bm, bn = 512, 1024
    mh = s // bm // NCORES
    return pl.pallas_call(
        _proj_kernel,
        grid=(NCORES, mh, n // bn),
        in_specs=[
            pl.BlockSpec((bm, dm), lambda c, i, j: (c * mh + i, 0)),
            pl.BlockSpec((dm, bn), lambda c, i, j: (0, j)),
        ],
        out_specs=pl.BlockSpec((bm, bn), lambda c, i, j: (c * mh + i, j)),
        out_shape=jax.ShapeDtypeStruct((s, n), jnp.float32),
        compiler_params=pltpu.CompilerParams(
            dimension_semantics=("core_parallel", "parallel", "parallel")),
        name="proj_xbc",
    )(x0, w1t_bf)


def _ssd_kernel(nheads,
                xb2_ref, w1_ref, wdt_ref, wdtp_ref,
                cw_ref, cb_ref, ap_ref, dtb_ref, app_ref, dtbp_ref,
                y_ref, inv_ref, state_ref, carry_ref, tail_ref):
    i = pl.program_id(1)
    f32 = jnp.float32
    bf = jnp.bfloat16
    nh_loc = nheads // NCORES
    part = nh_loc * HP
    srows = xb2_ref.shape[1]
    slab = w1_ref.shape[1]

    @pl.when(i == 0)
    def _init():
        state_ref[...] = jnp.zeros_like(state_ref)
        carry_ref[...] = jnp.zeros_like(carry_ref)

    nbl = xb2_ref.shape[0]
    rows = nbl * srows
    xall = xb2_ref[...].reshape(rows, xb2_ref.shape[2]).astype(bf)

    cur = jax.lax.dot_general(
        xall[0:srows, :], w1_ref[...],
        dimension_numbers=(((1,), (0,)), ((), ())),
        preferred_element_type=f32)
    tail8 = jnp.where(i > 0, tail_ref[...], 0.0)
    tail_ref[...] = cur[srows - 8:srows, :]

    ext = jnp.concatenate([cur, tail8], axis=0).astype(bf)
    ie = jax.lax.broadcasted_iota(jnp.int32, (srows, srows + 8), 0)
    je = jax.lax.broadcasted_iota(jnp.int32, (srows, srows + 8), 1)
    conv = cur * cw_ref[3:4, :] + cb_ref[...]
    for k in (1, 2, 3):
        mk = (((je == ie - k) & (je < srows)) |
              ((je == srows + 8 - k + ie) & (ie < k)))
        shifted = jax.lax.dot_general(
            mk.astype(bf), ext, dimension_numbers=(((1,), (0,)), ((), ())),
            preferred_element_type=f32)
        conv += shifted * cw_ref[3 - k:4 - k, :]

    dtraw = jax.lax.dot_general(
        xall, wdt_ref[...], dimension_numbers=(((1,), (0,)), ((), ())),
        preferred_element_type=f32) + dtb_ref[...]
    dt = jnp.maximum(dtraw, 0.0) + jnp.log1p(jnp.exp(-jnp.abs(dtraw)))
    a_all = ap_ref[...] * dt

    ii = jax.lax.broadcasted_iota(jnp.int32, (rows, rows), 0)
    jj = jax.lax.broadcasted_iota(jnp.int32, (rows, rows), 1)
    blkmask = ((jj <= ii) & ((ii // srows) == (jj // srows))).astype(f32)
    ahi = a_all.astype(bf)
    alo = (a_all - ahi.astype(f32)).astype(bf)
    blk_bf = blkmask.astype(bf)
    cuml = (jax.lax.dot_general(
                blk_bf, ahi, dimension_numbers=(((1,), (0,)), ((), ())),
                preferred_element_type=f32) +
            jax.lax.dot_general(
                blk_bf, alo, dimension_numbers=(((1,), (0,)), ((), ())),
                preferred_element_type=f32))
    coff = carry_ref[0:nbl, :]
    rsel = ((ii[:, 0:nbl] // srows) ==
            jax.lax.broadcasted_iota(jnp.int32, (rows, nbl), 1)).astype(f32)
    cuma = cuml + jax.lax.dot_general(
        rsel, coff, dimension_numbers=(((1,), (0,)), ((), ())),
        preferred_element_type=f32, precision=HIGH)
    en = jnp.exp(-cuma)
    inner = jax.lax.dot_general(
        blk_bf, en.astype(bf),
        dimension_numbers=(((1,), (0,)), ((), ())),
        preferred_element_type=f32)
    inner += jax.lax.dot_general(
        rsel, carry_ref[4:4 + nbl, :],
        dimension_numbers=(((1,), (0,)), ((), ())),
        preferred_element_type=f32, precision=HIGH)
    inv_ref[...] = (1.0 / (jnp.exp(cuma) * inner)).reshape(nbl, srows, nheads)
    newoff = jnp.concatenate(
        [cuma[b * srows + srows - 1:b * srows + srows, :] for b in range(nbl)],
        axis=0)
    segsum = jnp.concatenate(
        [jnp.sum(en[b * srows:(b + 1) * srows, :], axis=0, keepdims=True)
         for b in range(nbl)], axis=0)
    carry_ref[0:nbl, :] = newoff
    carry_ref[4:4 + nbl, :] = carry_ref[4:4 + nbl, :] + segsum

    nsub = srows // CHUNK
    x0 = xall[0:srows, :]
    dtraw0 = jax.lax.dot_general(
        x0, wdtp_ref[...], dimension_numbers=(((1,), (0,)), ((), ())),
        preferred_element_type=f32) + dtbp_ref[...]
    dt0 = jnp.maximum(dtraw0, 0.0) + jnp.log1p(jnp.exp(-jnp.abs(dtraw0)))
    a0 = (app_ref[...] * dt0).astype(bf)
    si = jax.lax.broadcasted_iota(jnp.int32, (srows, srows), 0)
    sj = jax.lax.broadcasted_iota(jnp.int32, (srows, srows), 1)
    btri = (sj <= si) & ((si // CHUNK) == (sj // CHUNK))
    cum0 = jax.lax.dot_general(
        btri.astype(bf), a0, dimension_numbers=(((1,), (0,)), ((), ())),
        preferred_element_type=f32)
    u = jnp.exp(cum0)
    v = jnp.exp(-cum0)
    li = jax.lax.broadcasted_iota(jnp.int32, (CHUNK, CHUNK), 0)
    lj = jax.lax.broadcasted_iota(jnp.int32, (CHUNK, CHUNK), 1)
    ltri = lj <= li
    for h in range(nh_loc):
        sl = slice(h * HP, (h + 1) * HP)
        sh = state_ref[sl, :]
        ys = []
        for sck in range(nsub):
            rs = slice(sck * CHUNK, (sck + 1) * CHUNK)
            ucol = u[rs, h:h + 1]
            vcol = v[rs, h:h + 1]
            ct = (conv[rs, sl] * ucol).astype(bf)
            bv = (conv[rs, part + h * HP:part + (h + 1) * HP]
                  * vcol).astype(bf)
            xh = conv[rs, 2 * part + h * HP:2 * part + (h + 1) * HP
                      ].astype(bf)
            g = jax.lax.dot_general(
                ct, bv, dimension_numbers=(((1,), (1,)), ((), ())),
                preferred_element_type=f32)
            gm = jnp.where(ltri, g, 0.0).astype(bf)
            ys.append(jax.lax.dot_general(
                jnp.concatenate([gm, ct], axis=1),
                jnp.concatenate([xh, sh.astype(bf)], axis=0),
                dimension_numbers=(((1,), (0,)), ((), ())),
                preferred_element_type=f32))
            sc = jax.lax.dot_general(
                bv, xh, dimension_numbers=(((0,), (0,)), ((), ())),
                preferred_element_type=f32)
            sh = (sh + sc) * u[sck * CHUNK + CHUNK - 1:
                               sck * CHUNK + CHUNK, h:h + 1]
        state_ref[sl, :] = sh
        y_ref[:, sl] = ys[0] if nsub == 1 else jnp.concatenate(ys, axis=0)


def _ssd(w1_bf, x, wdt_bf, wdtp_bf, cwp, cbp, ap_row, dtb_row,
         app_row, dtbp_row):
    nb, s, dm = x.shape
    dcc = w1_bf.shape[1]
    nheads = ap_row.shape[1]
    slab = dcc // NCORES
    nbl = nb // NCORES
    srows = 2 * CHUNK
    nsteps = s // srows
    kfn = functools.partial(_ssd_kernel, nheads)
    return pl.pallas_call(
        kfn,
        grid=(NCORES, nsteps),
        in_specs=[
            pl.BlockSpec((nbl, srows, dm), lambda c, i: (c, i, 0)),
            pl.BlockSpec((dm, slab), lambda c, i: (0, c)),
            pl.BlockSpec((dm, nheads), lambda c, i: (0, 0)),
            pl.BlockSpec((dm, 128), lambda c, i: (0, c)),
            pl.BlockSpec((D_CONV, slab), lambda c, i: (0, c)),
            pl.BlockSpec((1, slab), lambda c, i: (0, c)),
            pl.BlockSpec((1, nheads), lambda c, i: (0, 0)),
            pl.BlockSpec((1, nheads), lambda c, i: (0, 0)),
            pl.BlockSpec((1, 128), lambda c, i: (0, c)),
            pl.BlockSpec((1, 128), lambda c, i: (0, c)),
        ],
        out_specs=[
            pl.BlockSpec((srows, slab // 3), lambda c, i: (i, c)),
            pl.BlockSpec((nbl, srows, nheads), lambda c, i: (c, i, 0)),
        ],
        out_shape=[
            jax.ShapeDtypeStruct((s, dcc // 3), jnp.float32),
            jax.ShapeDtypeStruct((nb, s, nheads), jnp.float32),
        ],
        scratch_shapes=[
            pltpu.VMEM((slab // 3, HP), jnp.float32),
            pltpu.VMEM((8, nheads), jnp.float32),
            pltpu.VMEM((8, slab), jnp.float32),
        ],
        compiler_params=pltpu.CompilerParams(
            dimension_semantics=("core_parallel", "arbitrary")),
        name="proj_conv_ssd_norm",
    )(x, w1_bf, wdt_bf, wdtp_bf, cwp, cbp, ap_row, dtb_row,
      app_row, dtbp_row)


def _out_kernel(nheads, y_ref, inv_ref, w_ref, o_ref):
    f32 = jnp.float32
    bm = y_ref.shape[0]
    di = y_ref.shape[1]
    hp = di // nheads
    inv = inv_ref[...].reshape(bm, nheads)
    invx = jnp.concatenate(
        [jnp.broadcast_to(inv[:, h:h + 1], (bm, hp)) for h in range(nheads)],
        axis=1)
    z = (y_ref[...] * invx).astype(jnp.bfloat16)
    o = jax.lax.dot_general(
        z, w_ref[...], dimension_numbers=(((1,), (0,)), ((), ())),
        preferred_element_type=f32)
    o_ref[...] = o.reshape(1, bm, o.shape[1])


def _out_proj(y0, invn, wot_bf):
    nb, s, nheads = invn.shape
    di = y0.shape[1]
    dm = wot_bf.shape[1]
    bm = 1024
    nbl = nb // NCORES
    kfn = functools.partial(_out_kernel, nheads)
    return pl.pallas_call(
        kfn,
        grid=(NCORES, s // bm, nbl),
        in_specs=[
            pl.BlockSpec((bm, di), lambda c, m, b: (m, 0)),
            pl.BlockSpec((1, bm, nheads),
                         lambda c, m, b: (c * nbl + b, m, 0)),
            pl.BlockSpec((di, dm), lambda c, m, b: (0, 0)),
        ],
        out_specs=pl.BlockSpec((1, bm, dm),
                               lambda c, m, b: (c * nbl + b, m, 0)),
        out_shape=jax.ShapeDtypeStruct((nb, s, dm), jnp.float32),
        compiler_params=pltpu.CompilerParams(
            dimension_semantics=("core_parallel", "parallel", "parallel")),
        name="scale_outproj",
    )(y0, invn, wot_bf)


def kernel(x, W_in, conv_w, conv_b, A_param, dt_bias, W_out):
    nb, s, dm = x.shape
    nheads = A_param.shape[0]
    dcc = conv_w.shape[0]
    nh_loc = nheads // NCORES

    def permute_cols(a):
        lead = a.shape[:-1]
        ap = a.reshape(*lead, 3, NCORES, nh_loc, HP)
        ap = jnp.moveaxis(ap, -4, -3)
        return ap.reshape(*lead, dcc)


    w1t_bf = permute_cols(W_in[:dcc].T).astype(jnp.bfloat16)
    wdt = W_in[dcc:].T
    wdt_bf = wdt.astype(jnp.bfloat16)
    wdtp = jnp.zeros((dm, NCORES * 128), jnp.float32)
    app_row = jnp.zeros((1, NCORES * 128), jnp.float32)
    dtbp_row = jnp.zeros((1, NCORES * 128), jnp.float32)
    for c in range(NCORES):
        hs = slice(c * nh_loc, (c + 1) * nh_loc)
        cs = slice(c * 128, c * 128 + nh_loc)
        wdtp = wdtp.at[:, cs].set(wdt[:, hs])
        app_row = app_row.at[0, cs].set(A_param[hs])
        dtbp_row = dtbp_row.at[0, cs].set(dt_bias[hs])
    wdtp_bf = wdtp.astype(jnp.bfloat16)
    cwp = permute_cols(conv_w.T)
    cbp = permute_cols(conv_b.reshape(1, dcc))
    ap_row = A_param.reshape(1, nheads)
    dtb_row = dt_bias.reshape(1, nheads)
    wot_bf = W_out.T.astype(jnp.bfloat16)

    y0p, invn = _ssd(w1t_bf, x, wdt_bf, wdtp_bf, cwp, cbp, ap_row, dtb_row,
                     app_row, dtbp_row)
    return _out_proj(y0p, invn, wot_bf)


# speedup vs baseline: 1.0984x; 1.0040x over previous
import functools

import jax
import jax.numpy as jnp
from jax.experimental import pallas as pl
from jax.experimental.pallas import tpu as pltpu

CHUNK = 64
D_CONV = 4
NCORES = 1
HP = 64
HIGH = jax.lax.Precision.HIGHEST


def _proj_kernel(x_ref, w_ref, o_ref):
    xb = x_ref[...].astype(jnp.bfloat16)
    o_ref[...] = jax.lax.dot_general(
        xb, w_ref[...],
        dimension_numbers=(((1,), (0,)), ((), ())),
        preferred_element_type=jnp.float32)


def _proj_xbc(x0, w1t_bf):
    s, dm = x0.shape
    n = w1t_bf.shape[1]
    bm, bn = 512, 1024
    mh = s // bm // NCORES
    return pl.pallas_call(
        _proj_kernel,
        grid=(NCORES, mh, n // bn),
        in_specs=[
            pl.BlockSpec((bm, dm), lambda c, i, j: (c * mh + i, 0)),
            pl.BlockSpec((dm, bn), lambda c, i, j: (0, j)),
        ],
        out_specs=pl.BlockSpec((bm, bn), lambda c, i, j: (c * mh + i, j)),
        out_shape=jax.ShapeDtypeStruct((s, n), jnp.float32),
        compiler_params=pltpu.CompilerParams(
            dimension_semantics=("core_parallel", "parallel", "parallel")),
        name="proj_xbc",
    )(x0, w1t_bf)


def _ssd_kernel(nheads,
                xb2_ref, w1_ref, wdt_ref, wdtp_ref,
                cw_ref, cb_ref, ap_ref, dtb_ref, app_ref, dtbp_ref,
                y_ref, inv_ref, state_ref, carry_ref, tail_ref):
    i = pl.program_id(1)
    f32 = jnp.float32
    bf = jnp.bfloat16
    nh_loc = nheads // NCORES
    part = nh_loc * HP
    srows = xb2_ref.shape[1]
    slab = w1_ref.shape[1]

    @pl.when(i == 0)
    def _init():
        state_ref[...] = jnp.zeros_like(state_ref)
        carry_ref[...] = jnp.zeros_like(carry_ref)

    nbl = xb2_ref.shape[0]
    rows = nbl * srows
    xall = xb2_ref[...].reshape(rows, xb2_ref.shape[2]).astype(bf)

    cur = jax.lax.dot_general(
        xall[0:srows, :], w1_ref[...],
        dimension_numbers=(((1,), (0,)), ((), ())),
        preferred_element_type=f32)
    tail8 = jnp.where(i > 0, tail_ref[...], 0.0)
    tail_ref[...] = cur[srows - 8:srows, :]

    ext = jnp.concatenate([cur, tail8], axis=0).astype(bf)
    ie = jax.lax.broadcasted_iota(jnp.int32, (srows, srows + 8), 0)
    je = jax.lax.broadcasted_iota(jnp.int32, (srows, srows + 8), 1)
    conv = cur * cw_ref[3:4, :] + cb_ref[...]
    for k in (1, 2, 3):
        mk = (((je == ie - k) & (je < srows)) |
              ((je == srows + 8 - k + ie) & (ie < k)))
        shifted = jax.lax.dot_general(
            mk.astype(bf), ext, dimension_numbers=(((1,), (0,)), ((), ())),
            preferred_element_type=f32)
        conv += shifted * cw_ref[3 - k:4 - k, :]

    dtraw = jax.lax.dot_general(
        xall, wdt_ref[...], dimension_numbers=(((1,), (0,)), ((), ())),
        preferred_element_type=f32) + dtb_ref[...]
    dt = jnp.maximum(dtraw, 0.0) + jnp.log1p(jnp.exp(-jnp.abs(dtraw)))
    a_all = ap_ref[...] * dt

    ii = jax.lax.broadcasted_iota(jnp.int32, (rows, rows), 0)
    jj = jax.lax.broadcasted_iota(jnp.int32, (rows, rows), 1)
    blkmask = ((jj <= ii) & ((ii // srows) == (jj // srows))).astype(f32)
    ahi = a_all.astype(bf)
    alo = (a_all - ahi.astype(f32)).astype(bf)
    blk_bf = blkmask.astype(bf)
    cuml = (jax.lax.dot_general(
                blk_bf, ahi, dimension_numbers=(((1,), (0,)), ((), ())),
                preferred_element_type=f32) +
            jax.lax.dot_general(
                blk_bf, alo, dimension_numbers=(((1,), (0,)), ((), ())),
                preferred_element_type=f32))
    coff = carry_ref[0:nbl, :]
    rsel = ((ii[:, 0:nbl] // srows) ==
            jax.lax.broadcasted_iota(jnp.int32, (rows, nbl), 1)).astype(f32)
    cuma = cuml + jax.lax.dot_general(
        rsel, coff, dimension_numbers=(((1,), (0,)), ((), ())),
        preferred_element_type=f32, precision=HIGH)
    en = jnp.exp(-cuma)
    inner = jax.lax.dot_general(
        blk_bf, en.astype(bf),
        dimension_numbers=(((1,), (0,)), ((), ())),
        preferred_element_type=f32)
    inner += jax.lax.dot_general(
        rsel, carry_ref[4:4 + nbl, :],
        dimension_numbers=(((1,), (0,)), ((), ())),
        preferred_element_type=f32, precision=HIGH)
    inv_ref[...] = (1.0 / (jnp.exp(cuma) * inner)).reshape(nbl, srows, nheads)
    newoff = jnp.concatenate(
        [cuma[b * srows + srows - 1:b * srows + srows, :] for b in range(nbl)],
        axis=0)
    segsum = jnp.concatenate(
        [jnp.sum(en[b * srows:(b + 1) * srows, :], axis=0, keepdims=True)
         for b in range(nbl)], axis=0)
    carry_ref[0:nbl, :] = newoff
    carry_ref[4:4 + nbl, :] = carry_ref[4:4 + nbl, :] + segsum

    nsub = srows // CHUNK
    if NCORES == 1:
        a0 = a_all[0:srows, :].astype(bf)
    else:
        x0 = xall[0:srows, :]
        dtraw0 = jax.lax.dot_general(
            x0, wdtp_ref[...], dimension_numbers=(((1,), (0,)), ((), ())),
            preferred_element_type=f32) + dtbp_ref[...]
        dt0 = (jnp.maximum(dtraw0, 0.0)
               + jnp.log1p(jnp.exp(-jnp.abs(dtraw0))))
        a0 = (app_ref[...] * dt0).astype(bf)
    si = jax.lax.broadcasted_iota(jnp.int32, (srows, srows), 0)
    sj = jax.lax.broadcasted_iota(jnp.int32, (srows, srows), 1)
    btri = (sj <= si) & ((si // CHUNK) == (sj // CHUNK))
    cum0 = jax.lax.dot_general(
        btri.astype(bf), a0, dimension_numbers=(((1,), (0,)), ((), ())),
        preferred_element_type=f32)
    u = jnp.exp(cum0)
    v = jnp.exp(-cum0)
    nl = u.shape[1]
    gh = jax.lax.broadcasted_iota(jnp.int32, (nl, part), 0)
    gc = jax.lax.broadcasted_iota(jnp.int32, (nl, part), 1)
    esel = ((gc // HP) == gh).astype(bf)
    uexp = jax.lax.dot_general(
        u.astype(bf), esel, dimension_numbers=(((1,), (0,)), ((), ())),
        preferred_element_type=f32)
    vexp = jax.lax.dot_general(
        v.astype(bf), esel, dimension_numbers=(((1,), (0,)), ((), ())),
        preferred_element_type=f32)
    ctall = (conv[:, 0:part] * uexp).astype(bf)
    bvall = (conv[:, part:2 * part] * vexp).astype(bf)
    xhall = conv[:, 2 * part:3 * part].astype(bf)
    li = jax.lax.broadcasted_iota(jnp.int32, (CHUNK, CHUNK), 0)
    lj = jax.lax.broadcasted_iota(jnp.int32, (CHUNK, CHUNK), 1)
    ltri = lj <= li
    for h in range(nh_loc):
        sl = slice(h * HP, (h + 1) * HP)
        sh = state_ref[sl, :]
        ys = []
        for sck in range(nsub):
            rs = slice(sck * CHUNK, (sck + 1) * CHUNK)
            ct = ctall[rs, sl]
            bv = bvall[rs, sl]
            xh = xhall[rs, sl]
            g = jax.lax.dot_general(
                ct, bv, dimension_numbers=(((1,), (1,)), ((), ())),
                preferred_element_type=f32)
            gm = jnp.where(ltri, g, 0.0).astype(bf)
            ys.append(jax.lax.dot_general(
                jnp.concatenate([gm, ct], axis=1),
                jnp.concatenate([xh, sh.astype(bf)], axis=0),
                dimension_numbers=(((1,), (0,)), ((), ())),
                preferred_element_type=f32))
            sc = jax.lax.dot_general(
                bv, xh, dimension_numbers=(((0,), (0,)), ((), ())),
                preferred_element_type=f32)
            sh = (sh + sc) * u[sck * CHUNK + CHUNK - 1:
                               sck * CHUNK + CHUNK, h:h + 1]
        state_ref[sl, :] = sh
        y_ref[:, sl] = ys[0] if nsub == 1 else jnp.concatenate(ys, axis=0)


def _ssd(w1_bf, x, wdt_bf, wdtp_bf, cwp, cbp, ap_row, dtb_row,
         app_row, dtbp_row):
    nb, s, dm = x.shape
    dcc = w1_bf.shape[1]
    nheads = ap_row.shape[1]
    slab = dcc // NCORES
    nbl = nb // NCORES
    srows = 2 * CHUNK
    nsteps = s // srows
    kfn = functools.partial(_ssd_kernel, nheads)
    return pl.pallas_call(
        kfn,
        grid=(NCORES, nsteps),
        in_specs=[
            pl.BlockSpec((nbl, srows, dm), lambda c, i: (c, i, 0)),
            pl.BlockSpec((dm, slab), lambda c, i: (0, c)),
            pl.BlockSpec((dm, nheads), lambda c, i: (0, 0)),
            pl.BlockSpec((dm, 128), lambda c, i: (0, c)),
            pl.BlockSpec((D_CONV, slab), lambda c, i: (0, c)),
            pl.BlockSpec((1, slab), lambda c, i: (0, c)),
            pl.BlockSpec((1, nheads), lambda c, i: (0, 0)),
            pl.BlockSpec((1, nheads), lambda c, i: (0, 0)),
            pl.BlockSpec((1, 128), lambda c, i: (0, c)),
            pl.BlockSpec((1, 128), lambda c, i: (0, c)),
        ],
        out_specs=[
            pl.BlockSpec((srows, slab // 3), lambda c, i: (i, c)),
            pl.BlockSpec((nbl, srows, nheads), lambda c, i: (c, i, 0)),
        ],
        out_shape=[
            jax.ShapeDtypeStruct((s, dcc // 3), jnp.float32),
            jax.ShapeDtypeStruct((nb, s, nheads), jnp.float32),
        ],
        scratch_shapes=[
            pltpu.VMEM((slab // 3, HP), jnp.float32),
            pltpu.VMEM((8, nheads), jnp.float32),
            pltpu.VMEM((8, slab), jnp.float32),
        ],
        compiler_params=pltpu.CompilerParams(
            dimension_semantics=("core_parallel", "arbitrary")),
        name="proj_conv_ssd_norm",
    )(x, w1_bf, wdt_bf, wdtp_bf, cwp, cbp, ap_row, dtb_row,
      app_row, dtbp_row)


def _out_kernel(nheads, y_ref, inv_ref, w_ref, o_ref):
    f32 = jnp.float32
    bm = y_ref.shape[0]
    di = y_ref.shape[1]
    hp = di // nheads
    inv = inv_ref[...].reshape(bm, nheads)
    invx = jnp.concatenate(
        [jnp.broadcast_to(inv[:, h:h + 1], (bm, hp)) for h in range(nheads)],
        axis=1)
    z = (y_ref[...] * invx).astype(jnp.bfloat16)
    o = jax.lax.dot_general(
        z, w_ref[...], dimension_numbers=(((1,), (0,)), ((), ())),
        preferred_element_type=f32)
    o_ref[...] = o.reshape(1, bm, o.shape[1])


def _out_proj(y0, invn, wot_bf):
    nb, s, nheads = invn.shape
    di = y0.shape[1]
    dm = wot_bf.shape[1]
    bm = 1024
    nbl = nb // NCORES
    kfn = functools.partial(_out_kernel, nheads)
    return pl.pallas_call(
        kfn,
        grid=(NCORES, s // bm, nbl),
        in_specs=[
            pl.BlockSpec((bm, di), lambda c, m, b: (m, 0)),
            pl.BlockSpec((1, bm, nheads),
                         lambda c, m, b: (c * nbl + b, m, 0)),
            pl.BlockSpec((di, dm), lambda c, m, b: (0, 0)),
        ],
        out_specs=pl.BlockSpec((1, bm, dm),
                               lambda c, m, b: (c * nbl + b, m, 0)),
        out_shape=jax.ShapeDtypeStruct((nb, s, dm), jnp.float32),
        compiler_params=pltpu.CompilerParams(
            dimension_semantics=("core_parallel", "parallel", "parallel")),
        name="scale_outproj",
    )(y0, invn, wot_bf)


def kernel(x, W_in, conv_w, conv_b, A_param, dt_bias, W_out):
    nb, s, dm = x.shape
    nheads = A_param.shape[0]
    dcc = conv_w.shape[0]
    nh_loc = nheads // NCORES

    def permute_cols(a):
        lead = a.shape[:-1]
        ap = a.reshape(*lead, 3, NCORES, nh_loc, HP)
        ap = jnp.moveaxis(ap, -4, -3)
        return ap.reshape(*lead, dcc)


    w1t_bf = permute_cols(W_in[:dcc].T).astype(jnp.bfloat16)
    wdt = W_in[dcc:].T
    wdt_bf = wdt.astype(jnp.bfloat16)
    wdtp = jnp.zeros((dm, NCORES * 128), jnp.float32)
    app_row = jnp.zeros((1, NCORES * 128), jnp.float32)
    dtbp_row = jnp.zeros((1, NCORES * 128), jnp.float32)
    for c in range(NCORES):
        hs = slice(c * nh_loc, (c + 1) * nh_loc)
        cs = slice(c * 128, c * 128 + nh_loc)
        wdtp = wdtp.at[:, cs].set(wdt[:, hs])
        app_row = app_row.at[0, cs].set(A_param[hs])
        dtbp_row = dtbp_row.at[0, cs].set(dt_bias[hs])
    wdtp_bf = wdtp.astype(jnp.bfloat16)
    cwp = permute_cols(conv_w.T)
    cbp = permute_cols(conv_b.reshape(1, dcc))
    ap_row = A_param.reshape(1, nheads)
    dtb_row = dt_bias.reshape(1, nheads)
    wot_bf = W_out.T.astype(jnp.bfloat16)

    y0p, invn = _ssd(w1t_bf, x, wdt_bf, wdtp_bf, cwp, cbp, ap_row, dtb_row,
                     app_row, dtbp_row)
    return _out_proj(y0p, invn, wot_bf)


# speedup vs baseline: 1.1361x; 1.0343x over previous
import functools

import jax
import jax.numpy as jnp
from jax.experimental import pallas as pl
from jax.experimental.pallas import tpu as pltpu

CHUNK = 64
D_CONV = 4
NCORES = 1
HP = 64
HIGH = jax.lax.Precision.HIGHEST


def _proj_kernel(x_ref, w_ref, o_ref):
    xb = x_ref[...].astype(jnp.bfloat16)
    o_ref[...] = jax.lax.dot_general(
        xb, w_ref[...],
        dimension_numbers=(((1,), (0,)), ((), ())),
        preferred_element_type=jnp.float32)


def _proj_xbc(x0, w1t_bf):
    s, dm = x0.shape
    n = w1t_bf.shape[1]
    bm, bn = 512, 1024
    mh = s // bm // NCORES
    return pl.pallas_call(
        _proj_kernel,
        grid=(NCORES, mh, n // bn),
        in_specs=[
            pl.BlockSpec((bm, dm), lambda c, i, j: (c * mh + i, 0)),
            pl.BlockSpec((dm, bn), lambda c, i, j: (0, j)),
        ],
        out_specs=pl.BlockSpec((bm, bn), lambda c, i, j: (c * mh + i, j)),
        out_shape=jax.ShapeDtypeStruct((s, n), jnp.float32),
        compiler_params=pltpu.CompilerParams(
            dimension_semantics=("core_parallel", "parallel", "parallel")),
        name="proj_xbc",
    )(x0, w1t_bf)


def _ssd_kernel(nheads,
                xb2_ref, w1_ref, wdt_ref, wdtp_ref,
                cw_ref, cb_ref, ap_ref, dtb_ref, app_ref, dtbp_ref,
                y_ref, inv_ref, state_ref, carry_ref, tail_ref,
                conv_ref, bmask_ref):
    i = pl.program_id(1)
    f32 = jnp.float32
    bf = jnp.bfloat16
    nh_loc = nheads // NCORES
    part = nh_loc * HP
    srows = xb2_ref.shape[1]
    slab = w1_ref.shape[1]

    nbl = xb2_ref.shape[0]
    rows = nbl * srows

    @pl.when(i == 0)
    def _init():
        state_ref[...] = jnp.zeros_like(state_ref)
        carry_ref[...] = jnp.zeros_like(carry_ref)
        mi = jax.lax.broadcasted_iota(jnp.int32, (rows, rows), 0)
        mj = jax.lax.broadcasted_iota(jnp.int32, (rows, rows), 1)
        bmask_ref[...] = ((mj <= mi) &
                          ((mi // srows) == (mj // srows))).astype(bf)
    xall = xb2_ref[...].reshape(rows, xb2_ref.shape[2]).astype(bf)

    cur = jax.lax.dot_general(
        xall[0:srows, :], w1_ref[...],
        dimension_numbers=(((1,), (0,)), ((), ())),
        preferred_element_type=f32)
    tail8 = jnp.where(i > 0, tail_ref[...], 0.0)
    tail_ref[...] = cur[srows - 8:srows, :]

    ext = jnp.concatenate([cur, tail8], axis=0).astype(bf)
    ie = jax.lax.broadcasted_iota(jnp.int32, (srows, srows + 8), 0)
    je = jax.lax.broadcasted_iota(jnp.int32, (srows, srows + 8), 1)
    conv = cur * cw_ref[3:4, :] + cb_ref[...]
    for k in (1, 2, 3):
        mk = (((je == ie - k) & (je < srows)) |
              ((je == srows + 8 - k + ie) & (ie < k)))
        shifted = jax.lax.dot_general(
            mk.astype(bf), ext, dimension_numbers=(((1,), (0,)), ((), ())),
            preferred_element_type=f32)
        conv += shifted * cw_ref[3 - k:4 - k, :]
    conv_ref[...] = conv

    dtraw = jax.lax.dot_general(
        xall, wdt_ref[...], dimension_numbers=(((1,), (0,)), ((), ())),
        preferred_element_type=f32) + dtb_ref[...]
    dt = jnp.maximum(dtraw, 0.0) + jnp.log1p(jnp.exp(-jnp.abs(dtraw)))
    a_all = ap_ref[...] * dt

    ahi = a_all.astype(bf)
    alo = (a_all - ahi.astype(f32)).astype(bf)
    blk_bf = bmask_ref[...]
    cuml = (jax.lax.dot_general(
                blk_bf, ahi, dimension_numbers=(((1,), (0,)), ((), ())),
                preferred_element_type=f32) +
            jax.lax.dot_general(
                blk_bf, alo, dimension_numbers=(((1,), (0,)), ((), ())),
                preferred_element_type=f32))
    coff = carry_ref[0:nbl, :]
    rsel = ((jax.lax.broadcasted_iota(jnp.int32, (rows, nbl), 0) // srows) ==
            jax.lax.broadcasted_iota(jnp.int32, (rows, nbl), 1)).astype(f32)
    cuma = cuml + jax.lax.dot_general(
        rsel, coff, dimension_numbers=(((1,), (0,)), ((), ())),
        preferred_element_type=f32, precision=HIGH)
    en = jnp.exp(-cuma)
    inner = jax.lax.dot_general(
        blk_bf, en.astype(bf),
        dimension_numbers=(((1,), (0,)), ((), ())),
        preferred_element_type=f32)
    inner += jax.lax.dot_general(
        rsel, carry_ref[4:4 + nbl, :],
        dimension_numbers=(((1,), (0,)), ((), ())),
        preferred_element_type=f32, precision=HIGH)
    inv_ref[...] = (1.0 / (jnp.exp(cuma) * inner)).reshape(nbl, srows, nheads)
    newoff = jnp.concatenate(
        [cuma[b * srows + srows - 1:b * srows + srows, :] for b in range(nbl)],
        axis=0)
    segsum = jnp.concatenate(
        [jnp.sum(en[b * srows:(b + 1) * srows, :], axis=0, keepdims=True)
         for b in range(nbl)], axis=0)
    carry_ref[0:nbl, :] = newoff
    carry_ref[4:4 + nbl, :] = carry_ref[4:4 + nbl, :] + segsum

    nsub = srows // CHUNK
    if NCORES == 1:
        a0 = a_all[0:srows, :].astype(bf)
    else:
        x0 = xall[0:srows, :]
        dtraw0 = jax.lax.dot_general(
            x0, wdtp_ref[...], dimension_numbers=(((1,), (0,)), ((), ())),
            preferred_element_type=f32) + dtbp_ref[...]
        dt0 = (jnp.maximum(dtraw0, 0.0)
               + jnp.log1p(jnp.exp(-jnp.abs(dtraw0))))
        a0 = (app_ref[...] * dt0).astype(bf)
    si = jax.lax.broadcasted_iota(jnp.int32, (srows, srows), 0)
    sj = jax.lax.broadcasted_iota(jnp.int32, (srows, srows), 1)
    btri = (sj <= si) & ((si // CHUNK) == (sj // CHUNK))
    cum0 = jax.lax.dot_general(
        btri.astype(bf), a0, dimension_numbers=(((1,), (0,)), ((), ())),
        preferred_element_type=f32)
    u = jnp.exp(cum0)
    v = jnp.exp(-cum0)
    li = jax.lax.broadcasted_iota(jnp.int32, (CHUNK, CHUNK), 0)
    lj = jax.lax.broadcasted_iota(jnp.int32, (CHUNK, CHUNK), 1)
    ltri = lj <= li
    for h in range(nh_loc):
        sl = slice(h * HP, (h + 1) * HP)
        sh = state_ref[sl, :]
        ys = []
        for sck in range(nsub):
            rs = slice(sck * CHUNK, (sck + 1) * CHUNK)
            ct = (conv_ref[rs, sl] * u[rs, h:h + 1]).astype(bf)
            bv = (conv_ref[rs, part + h * HP:part + (h + 1) * HP]
                  * v[rs, h:h + 1]).astype(bf)
            xh = conv_ref[rs, 2 * part + h * HP:2 * part + (h + 1) * HP
                          ].astype(bf)
            g = jax.lax.dot_general(
                ct, bv, dimension_numbers=(((1,), (1,)), ((), ())),
                preferred_element_type=f32)
            gm = jnp.where(ltri, g, 0.0).astype(bf)
            ys.append(jax.lax.dot_general(
                jnp.concatenate([gm, ct], axis=1),
                jnp.concatenate([xh, sh.astype(bf)], axis=0),
                dimension_numbers=(((1,), (0,)), ((), ())),
                preferred_element_type=f32))
            sc = jax.lax.dot_general(
                bv, xh, dimension_numbers=(((0,), (0,)), ((), ())),
                preferred_element_type=f32)
            sh = (sh + sc) * u[sck * CHUNK + CHUNK - 1:
                               sck * CHUNK + CHUNK, h:h + 1]
        state_ref[sl, :] = sh
        y_ref[:, sl] = ys[0] if nsub == 1 else jnp.concatenate(ys, axis=0)


def _ssd(w1_bf, x, wdt_bf, wdtp_bf, cwp, cbp, ap_row, dtb_row,
         app_row, dtbp_row):
    nb, s, dm = x.shape
    dcc = w1_bf.shape[1]
    nheads = ap_row.shape[1]
    slab = dcc // NCORES
    nbl = nb // NCORES
    srows = 2 * CHUNK
    nsteps = s // srows
    kfn = functools.partial(_ssd_kernel, nheads)
    return pl.pallas_call(
        kfn,
        grid=(NCORES, nsteps),
        in_specs=[
            pl.BlockSpec((nbl, srows, dm), lambda c, i: (c, i, 0)),
            pl.BlockSpec((dm, slab), lambda c, i: (0, c)),
            pl.BlockSpec((dm, nheads), lambda c, i: (0, 0)),
            pl.BlockSpec((dm, 128), lambda c, i: (0, c)),
            pl.BlockSpec((D_CONV, slab), lambda c, i: (0, c)),
            pl.BlockSpec((1, slab), lambda c, i: (0, c)),
            pl.BlockSpec((1, nheads), lambda c, i: (0, 0)),
            pl.BlockSpec((1, nheads), lambda c, i: (0, 0)),
            pl.BlockSpec((1, 128), lambda c, i: (0, c)),
            pl.BlockSpec((1, 128), lambda c, i: (0, c)),
        ],
        out_specs=[
            pl.BlockSpec((srows, slab // 3), lambda c, i: (i, c)),
            pl.BlockSpec((nbl, srows, nheads), lambda c, i: (c, i, 0)),
        ],
        out_shape=[
            jax.ShapeDtypeStruct((s, dcc // 3), jnp.float32),
            jax.ShapeDtypeStruct((nb, s, nheads), jnp.float32),
        ],
        scratch_shapes=[
            pltpu.VMEM((slab // 3, HP), jnp.float32),
            pltpu.VMEM((8, nheads), jnp.float32),
            pltpu.VMEM((8, slab), jnp.float32),
            pltpu.VMEM((srows, slab), jnp.float32),
            pltpu.VMEM((nbl * srows, nbl * srows), jnp.bfloat16),
        ],
        compiler_params=pltpu.CompilerParams(
            dimension_semantics=("core_parallel", "arbitrary")),
        name="proj_conv_ssd_norm",
    )(x, w1_bf, wdt_bf, wdtp_bf, cwp, cbp, ap_row, dtb_row,
      app_row, dtbp_row)


def _out_kernel(nheads, y_ref, inv_ref, w_ref, o_ref):
    f32 = jnp.float32
    bm = y_ref.shape[0]
    di = y_ref.shape[1]
    hp = di // nheads
    inv = inv_ref[...].reshape(bm, nheads)
    invx = jnp.concatenate(
        [jnp.broadcast_to(inv[:, h:h + 1], (bm, hp)) for h in range(nheads)],
        axis=1)
    z = (y_ref[...] * invx).astype(jnp.bfloat16)
    o = jax.lax.dot_general(
        z, w_ref[...], dimension_numbers=(((1,), (0,)), ((), ())),
        preferred_element_type=f32)
    o_ref[...] = o.reshape(1, bm, o.shape[1])


def _out_proj(y0, invn, wot_bf):
    nb, s, nheads = invn.shape
    di = y0.shape[1]
    dm = wot_bf.shape[1]
    bm = 1024
    nbl = nb // NCORES
    kfn = functools.partial(_out_kernel, nheads)
    return pl.pallas_call(
        kfn,
        grid=(NCORES, s // bm, nbl),
        in_specs=[
            pl.BlockSpec((bm, di), lambda c, m, b: (m, 0)),
            pl.BlockSpec((1, bm, nheads),
                         lambda c, m, b: (c * nbl + b, m, 0)),
            pl.BlockSpec((di, dm), lambda c, m, b: (0, 0)),
        ],
        out_specs=pl.BlockSpec((1, bm, dm),
                               lambda c, m, b: (c * nbl + b, m, 0)),
        out_shape=jax.ShapeDtypeStruct((nb, s, dm), jnp.float32),
        compiler_params=pltpu.CompilerParams(
            dimension_semantics=("core_parallel", "parallel", "parallel")),
        name="scale_outproj",
    )(y0, invn, wot_bf)


def kernel(x, W_in, conv_w, conv_b, A_param, dt_bias, W_out):
    nb, s, dm = x.shape
    nheads = A_param.shape[0]
    dcc = conv_w.shape[0]
    nh_loc = nheads // NCORES

    def permute_cols(a):
        lead = a.shape[:-1]
        ap = a.reshape(*lead, 3, NCORES, nh_loc, HP)
        ap = jnp.moveaxis(ap, -4, -3)
        return ap.reshape(*lead, dcc)


    w1t_bf = permute_cols(W_in[:dcc].T).astype(jnp.bfloat16)
    wdt = W_in[dcc:].T
    wdt_bf = wdt.astype(jnp.bfloat16)
    wdtp = jnp.zeros((dm, NCORES * 128), jnp.float32)
    app_row = jnp.zeros((1, NCORES * 128), jnp.float32)
    dtbp_row = jnp.zeros((1, NCORES * 128), jnp.float32)
    for c in range(NCORES):
        hs = slice(c * nh_loc, (c + 1) * nh_loc)
        cs = slice(c * 128, c * 128 + nh_loc)
        wdtp = wdtp.at[:, cs].set(wdt[:, hs])
        app_row = app_row.at[0, cs].set(A_param[hs])
        dtbp_row = dtbp_row.at[0, cs].set(dt_bias[hs])
    wdtp_bf = wdtp.astype(jnp.bfloat16)
    cwp = permute_cols(conv_w.T)
    cbp = permute_cols(conv_b.reshape(1, dcc))
    ap_row = A_param.reshape(1, nheads)
    dtb_row = dt_bias.reshape(1, nheads)
    wot_bf = W_out.T.astype(jnp.bfloat16)

    y0p, invn = _ssd(w1t_bf, x, wdt_bf, wdtp_bf, cwp, cbp, ap_row, dtb_row,
                     app_row, dtbp_row)
    return _out_proj(y0p, invn, wot_bf)


# speedup vs baseline: 1.1367x; 1.0005x over previous
import functools

import jax
import jax.numpy as jnp
from jax.experimental import pallas as pl
from jax.experimental.pallas import tpu as pltpu

CHUNK = 64
D_CONV = 4
NCORES = 1
HP = 64
HIGH = jax.lax.Precision.HIGHEST


def _proj_kernel(x_ref, w_ref, o_ref):
    xb = x_ref[...].astype(jnp.bfloat16)
    o_ref[...] = jax.lax.dot_general(
        xb, w_ref[...],
        dimension_numbers=(((1,), (0,)), ((), ())),
        preferred_element_type=jnp.float32)


def _proj_xbc(x0, w1t_bf):
    s, dm = x0.shape
    n = w1t_bf.shape[1]
    bm, bn = 512, 1024
    mh = s // bm // NCORES
    return pl.pallas_call(
        _proj_kernel,
        grid=(NCORES, mh, n // bn),
        in_specs=[
            pl.BlockSpec((bm, dm), lambda c, i, j: (c * mh + i, 0)),
            pl.BlockSpec((dm, bn), lambda c, i, j: (0, j)),
        ],
        out_specs=pl.BlockSpec((bm, bn), lambda c, i, j: (c * mh + i, j)),
        out_shape=jax.ShapeDtypeStruct((s, n), jnp.float32),
        compiler_params=pltpu.CompilerParams(
            dimension_semantics=("core_parallel", "parallel", "parallel")),
        name="proj_xbc",
    )(x0, w1t_bf)


def _ssd_kernel(nheads,
                xb2_ref, w1_ref, wdt_ref, wdtp_ref,
                cw_ref, cb_ref, ap_ref, dtb_ref, app_ref, dtbp_ref,
                y_ref, inv_ref, state_ref, carry_ref, tail_ref,
                conv_ref, bmask_ref, ext_ref):
    i = pl.program_id(1)
    f32 = jnp.float32
    bf = jnp.bfloat16
    nh_loc = nheads // NCORES
    part = nh_loc * HP
    srows = xb2_ref.shape[1]
    slab = w1_ref.shape[1]

    nbl = xb2_ref.shape[0]
    rows = nbl * srows

    @pl.when(i == 0)
    def _init():
        state_ref[...] = jnp.zeros_like(state_ref)
        carry_ref[...] = jnp.zeros_like(carry_ref)
        mi = jax.lax.broadcasted_iota(jnp.int32, (rows, rows), 0)
        mj = jax.lax.broadcasted_iota(jnp.int32, (rows, rows), 1)
        bmask_ref[...] = ((mj <= mi) &
                          ((mi // srows) == (mj // srows))).astype(bf)
    xall = xb2_ref[...].reshape(rows, xb2_ref.shape[2]).astype(bf)

    ext_ref[srows:srows + 8, :] = jnp.where(i > 0, tail_ref[...], 0.0)
    cur = jax.lax.dot_general(
        xall[0:srows, :], w1_ref[...],
        dimension_numbers=(((1,), (0,)), ((), ())),
        preferred_element_type=f32).astype(bf)
    ext_ref[0:srows, :] = cur
    tail_ref[...] = cur[srows - 8:srows, :]

    ie = jax.lax.broadcasted_iota(jnp.int32, (srows, srows + 8), 0)
    je = jax.lax.broadcasted_iota(jnp.int32, (srows, srows + 8), 1)
    conv = ext_ref[0:srows, :].astype(f32) * cw_ref[3:4, :] + cb_ref[...]
    for k in (1, 2, 3):
        mk = (((je == ie - k) & (je < srows)) |
              ((je == srows + 8 - k + ie) & (ie < k)))
        shifted = jax.lax.dot_general(
            mk.astype(bf), ext_ref[...],
            dimension_numbers=(((1,), (0,)), ((), ())),
            preferred_element_type=f32)
        conv += shifted * cw_ref[3 - k:4 - k, :]
    conv_ref[...] = conv

    dtraw = jax.lax.dot_general(
        xall, wdt_ref[...], dimension_numbers=(((1,), (0,)), ((), ())),
        preferred_element_type=f32) + dtb_ref[...]
    dt = jnp.maximum(dtraw, 0.0) + jnp.log1p(jnp.exp(-jnp.abs(dtraw)))
    a_all = ap_ref[...] * dt

    ahi = a_all.astype(bf)
    alo = (a_all - ahi.astype(f32)).astype(bf)
    blk_bf = bmask_ref[...]
    cuml = (jax.lax.dot_general(
                blk_bf, ahi, dimension_numbers=(((1,), (0,)), ((), ())),
                preferred_element_type=f32) +
            jax.lax.dot_general(
                blk_bf, alo, dimension_numbers=(((1,), (0,)), ((), ())),
                preferred_element_type=f32))
    coff = carry_ref[0:nbl, :]
    rsel = ((jax.lax.broadcasted_iota(jnp.int32, (rows, nbl), 0) // srows) ==
            jax.lax.broadcasted_iota(jnp.int32, (rows, nbl), 1)).astype(f32)
    cuma = cuml + jax.lax.dot_general(
        rsel, coff, dimension_numbers=(((1,), (0,)), ((), ())),
        preferred_element_type=f32, precision=HIGH)
    en = jnp.exp(-cuma)
    inner = jax.lax.dot_general(
        blk_bf, en.astype(bf),
        dimension_numbers=(((1,), (0,)), ((), ())),
        preferred_element_type=f32)
    inner += jax.lax.dot_general(
        rsel, carry_ref[4:4 + nbl, :],
        dimension_numbers=(((1,), (0,)), ((), ())),
        preferred_element_type=f32, precision=HIGH)
    inv_ref[...] = (1.0 / (jnp.exp(cuma) * inner)).reshape(nbl, srows, nheads)
    newoff = jnp.concatenate(
        [cuma[b * srows + srows - 1:b * srows + srows, :] for b in range(nbl)],
        axis=0)
    segsum = jnp.concatenate(
        [jnp.sum(en[b * srows:(b + 1) * srows, :], axis=0, keepdims=True)
         for b in range(nbl)], axis=0)
    carry_ref[0:nbl, :] = newoff
    carry_ref[4:4 + nbl, :] = carry_ref[4:4 + nbl, :] + segsum

    nsub = srows // CHUNK
    if NCORES == 1:
        a0 = a_all[0:srows, :].astype(bf)
    else:
        x0 = xall[0:srows, :]
        dtraw0 = jax.lax.dot_general(
            x0, wdtp_ref[...], dimension_numbers=(((1,), (0,)), ((), ())),
            preferred_element_type=f32) + dtbp_ref[...]
        dt0 = (jnp.maximum(dtraw0, 0.0)
               + jnp.log1p(jnp.exp(-jnp.abs(dtraw0))))
        a0 = (app_ref[...] * dt0).astype(bf)
    si = jax.lax.broadcasted_iota(jnp.int32, (srows, srows), 0)
    sj = jax.lax.broadcasted_iota(jnp.int32, (srows, srows), 1)
    btri = (sj <= si) & ((si // CHUNK) == (sj // CHUNK))
    cum0 = jax.lax.dot_general(
        btri.astype(bf), a0, dimension_numbers=(((1,), (0,)), ((), ())),
        preferred_element_type=f32)
    u = jnp.exp(cum0)
    v = jnp.exp(-cum0)
    li = jax.lax.broadcasted_iota(jnp.int32, (CHUNK, CHUNK), 0)
    lj = jax.lax.broadcasted_iota(jnp.int32, (CHUNK, CHUNK), 1)
    ltri = lj <= li
    for h in range(nh_loc):
        sl = slice(h * HP, (h + 1) * HP)
        sh = state_ref[sl, :]
        ys = []
        for sck in range(nsub):
            rs = slice(sck * CHUNK, (sck + 1) * CHUNK)
            ct = (conv_ref[rs, sl] * u[rs, h:h + 1]).astype(bf)
            bv = (conv_ref[rs, part + h * HP:part + (h + 1) * HP]
                  * v[rs, h:h + 1]).astype(bf)
            xh = conv_ref[rs, 2 * part + h * HP:2 * part + (h + 1) * HP
                          ].astype(bf)
            g = jax.lax.dot_general(
                ct, bv, dimension_numbers=(((1,), (1,)), ((), ())),
                preferred_element_type=f32)
            gm = jnp.where(ltri, g, 0.0).astype(bf)
            ys.append(jax.lax.dot_general(
                jnp.concatenate([gm, ct], axis=1),
                jnp.concatenate([xh, sh.astype(bf)], axis=0),
                dimension_numbers=(((1,), (0,)), ((), ())),
                preferred_element_type=f32))
            sc = jax.lax.dot_general(
                bv, xh, dimension_numbers=(((0,), (0,)), ((), ())),
                preferred_element_type=f32)
            sh = (sh + sc) * u[sck * CHUNK + CHUNK - 1:
                               sck * CHUNK + CHUNK, h:h + 1]
        state_ref[sl, :] = sh
        y_ref[:, sl] = ys[0] if nsub == 1 else jnp.concatenate(ys, axis=0)


def _ssd(w1_bf, x, wdt_bf, wdtp_bf, cwp, cbp, ap_row, dtb_row,
         app_row, dtbp_row):
    nb, s, dm = x.shape
    dcc = w1_bf.shape[1]
    nheads = ap_row.shape[1]
    slab = dcc // NCORES
    nbl = nb // NCORES
    srows = 2 * CHUNK
    nsteps = s // srows
    kfn = functools.partial(_ssd_kernel, nheads)
    return pl.pallas_call(
        kfn,
        grid=(NCORES, nsteps),
        in_specs=[
            pl.BlockSpec((nbl, srows, dm), lambda c, i: (c, i, 0)),
            pl.BlockSpec((dm, slab), lambda c, i: (0, c)),
            pl.BlockSpec((dm, nheads), lambda c, i: (0, 0)),
            pl.BlockSpec((dm, 128), lambda c, i: (0, c)),
            pl.BlockSpec((D_CONV, slab), lambda c, i: (0, c)),
            pl.BlockSpec((1, slab), lambda c, i: (0, c)),
            pl.BlockSpec((1, nheads), lambda c, i: (0, 0)),
            pl.BlockSpec((1, nheads), lambda c, i: (0, 0)),
            pl.BlockSpec((1, 128), lambda c, i: (0, c)),
            pl.BlockSpec((1, 128), lambda c, i: (0, c)),
        ],
        out_specs=[
            pl.BlockSpec((srows, slab // 3), lambda c, i: (i, c)),
            pl.BlockSpec((nbl, srows, nheads), lambda c, i: (c, i, 0)),
        ],
        out_shape=[
            jax.ShapeDtypeStruct((s, dcc // 3), jnp.float32),
            jax.ShapeDtypeStruct((nb, s, nheads), jnp.float32),
        ],
        scratch_shapes=[
            pltpu.VMEM((slab // 3, HP), jnp.float32),
            pltpu.VMEM((8, nheads), jnp.float32),
            pltpu.VMEM((8, slab), jnp.bfloat16),
            pltpu.VMEM((srows, slab), jnp.float32),
            pltpu.VMEM((nbl * srows, nbl * srows), jnp.bfloat16),
            pltpu.VMEM((srows + 8, slab), jnp.bfloat16),
        ],
        compiler_params=pltpu.CompilerParams(
            dimension_semantics=("core_parallel", "arbitrary")),
        name="proj_conv_ssd_norm",
    )(x, w1_bf, wdt_bf, wdtp_bf, cwp, cbp, ap_row, dtb_row,
      app_row, dtbp_row)


def _out_kernel(nheads, y_ref, inv_ref, w_ref, o_ref):
    f32 = jnp.float32
    bm = y_ref.shape[0]
    di = y_ref.shape[1]
    hp = di // nheads
    inv = inv_ref[...].reshape(bm, nheads)
    invx = jnp.concatenate(
        [jnp.broadcast_to(inv[:, h:h + 1], (bm, hp)) for h in range(nheads)],
        axis=1)
    z = (y_ref[...] * invx).astype(jnp.bfloat16)
    o = jax.lax.dot_general(
        z, w_ref[...], dimension_numbers=(((1,), (0,)), ((), ())),
        preferred_element_type=f32)
    o_ref[...] = o.reshape(1, bm, o.shape[1])


def _out_proj(y0, invn, wot_bf):
    nb, s, nheads = invn.shape
    di = y0.shape[1]
    dm = wot_bf.shape[1]
    bm = 1024
    nbl = nb // NCORES
    kfn = functools.partial(_out_kernel, nheads)
    return pl.pallas_call(
        kfn,
        grid=(NCORES, s // bm, nbl),
        in_specs=[
            pl.BlockSpec((bm, di), lambda c, m, b: (m, 0)),
            pl.BlockSpec((1, bm, nheads),
                         lambda c, m, b: (c * nbl + b, m, 0)),
            pl.BlockSpec((di, dm), lambda c, m, b: (0, 0)),
        ],
        out_specs=pl.BlockSpec((1, bm, dm),
                               lambda c, m, b: (c * nbl + b, m, 0)),
        out_shape=jax.ShapeDtypeStruct((nb, s, dm), jnp.float32),
        compiler_params=pltpu.CompilerParams(
            dimension_semantics=("core_parallel", "parallel", "parallel")),
        name="scale_outproj",
    )(y0, invn, wot_bf)


def kernel(x, W_in, conv_w, conv_b, A_param, dt_bias, W_out):
    nb, s, dm = x.shape
    nheads = A_param.shape[0]
    dcc = conv_w.shape[0]
    nh_loc = nheads // NCORES

    def permute_cols(a):
        lead = a.shape[:-1]
        ap = a.reshape(*lead, 3, NCORES, nh_loc, HP)
        ap = jnp.moveaxis(ap, -4, -3)
        return ap.reshape(*lead, dcc)


    w1t_bf = permute_cols(W_in[:dcc].T).astype(jnp.bfloat16)
    wdt = W_in[dcc:].T
    wdt_bf = wdt.astype(jnp.bfloat16)
    wdtp = jnp.zeros((dm, NCORES * 128), jnp.float32)
    app_row = jnp.zeros((1, NCORES * 128), jnp.float32)
    dtbp_row = jnp.zeros((1, NCORES * 128), jnp.float32)
    for c in range(NCORES):
        hs = slice(c * nh_loc, (c + 1) * nh_loc)
        cs = slice(c * 128, c * 128 + nh_loc)
        wdtp = wdtp.at[:, cs].set(wdt[:, hs])
        app_row = app_row.at[0, cs].set(A_param[hs])
        dtbp_row = dtbp_row.at[0, cs].set(dt_bias[hs])
    wdtp_bf = wdtp.astype(jnp.bfloat16)
    cwp = permute_cols(conv_w.T)
    cbp = permute_cols(conv_b.reshape(1, dcc))
    ap_row = A_param.reshape(1, nheads)
    dtb_row = dt_bias.reshape(1, nheads)
    wot_bf = W_out.T.astype(jnp.bfloat16)

    y0p, invn = _ssd(w1t_bf, x, wdt_bf, wdtp_bf, cwp, cbp, ap_row, dtb_row,
                     app_row, dtbp_row)
    return _out_proj(y0p, invn, wot_bf)


# speedup vs baseline: 1.1369x; 1.0002x over previous
import functools

import jax
import jax.numpy as jnp
from jax.experimental import pallas as pl
from jax.experimental.pallas import tpu as pltpu

CHUNK = 64
D_CONV = 4
NCORES = 1
HP = 64
HIGH = jax.lax.Precision.HIGHEST


def _ssd_kernel(nheads,
                xb2_ref, w1_ref, wdt_ref, wdtp_ref,
                cw_ref, cb_ref, ap_ref, dtb_ref, app_ref, dtbp_ref,
                y_ref, inv_ref, state_ref, carry_ref, tail_ref,
                conv_ref, bmask_ref, ext_ref):
    i = pl.program_id(1)
    f32 = jnp.float32
    bf = jnp.bfloat16
    nh_loc = nheads // NCORES
    part = nh_loc * HP
    srows = xb2_ref.shape[1]
    slab = w1_ref.shape[1]

    nbl = xb2_ref.shape[0]
    rows = nbl * srows

    @pl.when(i == 0)
    def _init():
        state_ref[...] = jnp.zeros_like(state_ref)
        carry_ref[...] = jnp.zeros_like(carry_ref)
        mi = jax.lax.broadcasted_iota(jnp.int32, (rows, rows), 0)
        mj = jax.lax.broadcasted_iota(jnp.int32, (rows, rows), 1)
        bmask_ref[...] = ((mj <= mi) &
                          ((mi // srows) == (mj // srows))).astype(bf)
    xall = xb2_ref[...].reshape(rows, xb2_ref.shape[2]).astype(bf)

    ext_ref[srows:srows + 8, :] = jnp.where(i > 0, tail_ref[...], 0.0)
    cur = jax.lax.dot_general(
        xall[0:srows, :], w1_ref[...],
        dimension_numbers=(((1,), (0,)), ((), ())),
        preferred_element_type=f32).astype(bf)
    ext_ref[0:srows, :] = cur
    tail_ref[...] = cur[srows - 8:srows, :]

    ie = jax.lax.broadcasted_iota(jnp.int32, (srows, srows + 8), 0)
    je = jax.lax.broadcasted_iota(jnp.int32, (srows, srows + 8), 1)
    conv = ext_ref[0:srows, :].astype(f32) * cw_ref[3:4, :] + cb_ref[...]
    for k in (1, 2, 3):
        mk = (((je == ie - k) & (je < srows)) |
              ((je == srows + 8 - k + ie) & (ie < k)))
        shifted = jax.lax.dot_general(
            mk.astype(bf), ext_ref[...],
            dimension_numbers=(((1,), (0,)), ((), ())),
            preferred_element_type=f32)
        conv += shifted * cw_ref[3 - k:4 - k, :]
    conv_ref[...] = conv

    dtraw = jax.lax.dot_general(
        xall, wdt_ref[...], dimension_numbers=(((1,), (0,)), ((), ())),
        preferred_element_type=f32) + dtb_ref[...]
    dt = jnp.maximum(dtraw, 0.0) + jnp.log1p(jnp.exp(-jnp.abs(dtraw)))
    a_all = ap_ref[...] * dt

    ahi = a_all.astype(bf)
    alo = (a_all - ahi.astype(f32)).astype(bf)
    blk_bf = bmask_ref[...]
    cuml = (jax.lax.dot_general(
                blk_bf, ahi, dimension_numbers=(((1,), (0,)), ((), ())),
                preferred_element_type=f32) +
            jax.lax.dot_general(
                blk_bf, alo, dimension_numbers=(((1,), (0,)), ((), ())),
                preferred_element_type=f32))
    coff = carry_ref[0:nbl, :]
    rsel = ((jax.lax.broadcasted_iota(jnp.int32, (rows, nbl), 0) // srows) ==
            jax.lax.broadcasted_iota(jnp.int32, (rows, nbl), 1)).astype(f32)
    cuma = cuml + jax.lax.dot_general(
        rsel, coff, dimension_numbers=(((1,), (0,)), ((), ())),
        preferred_element_type=f32, precision=HIGH)
    en = jnp.exp(-cuma)
    inner = jax.lax.dot_general(
        blk_bf, en.astype(bf),
        dimension_numbers=(((1,), (0,)), ((), ())),
        preferred_element_type=f32)
    inner += jax.lax.dot_general(
        rsel, carry_ref[4:4 + nbl, :],
        dimension_numbers=(((1,), (0,)), ((), ())),
        preferred_element_type=f32, precision=HIGH)
    inv_ref[...] = (1.0 / (jnp.exp(cuma) * inner)).reshape(nbl, srows, nheads)
    newoff = jnp.concatenate(
        [cuma[b * srows + srows - 1:b * srows + srows, :] for b in range(nbl)],
        axis=0)
    segsum = jnp.concatenate(
        [jnp.sum(en[b * srows:(b + 1) * srows, :], axis=0, keepdims=True)
         for b in range(nbl)], axis=0)
    carry_ref[0:nbl, :] = newoff
    carry_ref[4:4 + nbl, :] = carry_ref[4:4 + nbl, :] + segsum

    nsub = srows // CHUNK
    if NCORES == 1:
        a0 = a_all[0:srows, :].astype(bf)
    else:
        x0 = xall[0:srows, :]
        dtraw0 = jax.lax.dot_general(
            x0, wdtp_ref[...], dimension_numbers=(((1,), (0,)), ((), ())),
            preferred_element_type=f32) + dtbp_ref[...]
        dt0 = (jnp.maximum(dtraw0, 0.0)
               + jnp.log1p(jnp.exp(-jnp.abs(dtraw0))))
        a0 = (app_ref[...] * dt0).astype(bf)
    si = jax.lax.broadcasted_iota(jnp.int32, (srows, srows), 0)
    sj = jax.lax.broadcasted_iota(jnp.int32, (srows, srows), 1)
    btri = (sj <= si) & ((si // CHUNK) == (sj // CHUNK))
    cum0 = jax.lax.dot_general(
        btri.astype(bf), a0, dimension_numbers=(((1,), (0,)), ((), ())),
        preferred_element_type=f32)
    u = jnp.exp(cum0)
    v = jnp.exp(-cum0)
    li = jax.lax.broadcasted_iota(jnp.int32, (CHUNK, CHUNK), 0)
    lj = jax.lax.broadcasted_iota(jnp.int32, (CHUNK, CHUNK), 1)
    ltri = lj <= li
    for h in range(nh_loc):
        sl = slice(h * HP, (h + 1) * HP)
        sh = state_ref[sl, :]
        ys = []
        for sck in range(nsub):
            rs = slice(sck * CHUNK, (sck + 1) * CHUNK)
            ct = (conv_ref[rs, sl] * u[rs, h:h + 1]).astype(bf)
            bv = (conv_ref[rs, part + h * HP:part + (h + 1) * HP]
                  * v[rs, h:h + 1]).astype(bf)
            xh = conv_ref[rs, 2 * part + h * HP:2 * part + (h + 1) * HP
                          ].astype(bf)
            g = jax.lax.dot_general(
                ct, bv, dimension_numbers=(((1,), (1,)), ((), ())),
                preferred_element_type=f32)
            gm = jnp.where(ltri, g, 0.0).astype(bf)
            ys.append(jax.lax.dot_general(
                jnp.concatenate([gm, ct], axis=1),
                jnp.concatenate([xh, sh.astype(bf)], axis=0),
                dimension_numbers=(((1,), (0,)), ((), ())),
                preferred_element_type=f32))
            sc = jax.lax.dot_general(
                bv, xh, dimension_numbers=(((0,), (0,)), ((), ())),
                preferred_element_type=f32)
            sh = (sh + sc) * u[sck * CHUNK + CHUNK - 1:
                               sck * CHUNK + CHUNK, h:h + 1]
        state_ref[sl, :] = sh
        y_ref[:, sl] = ys[0] if nsub == 1 else jnp.concatenate(ys, axis=0)


def _ssd(w1_bf, x, wdt_bf, wdtp_bf, cwp, cbp, ap_row, dtb_row,
         app_row, dtbp_row):
    nb, s, dm = x.shape
    dcc = w1_bf.shape[1]
    nheads = ap_row.shape[1]
    slab = dcc // NCORES
    nbl = nb // NCORES
    srows = 2 * CHUNK
    nsteps = s // srows
    kfn = functools.partial(_ssd_kernel, nheads)
    return pl.pallas_call(
        kfn,
        grid=(NCORES, nsteps),
        in_specs=[
            pl.BlockSpec((nbl, srows, dm), lambda c, i: (c, i, 0)),
            pl.BlockSpec((dm, slab), lambda c, i: (0, c)),
            pl.BlockSpec((dm, nheads), lambda c, i: (0, 0)),
            pl.BlockSpec((dm, 128), lambda c, i: (0, c)),
            pl.BlockSpec((D_CONV, slab), lambda c, i: (0, c)),
            pl.BlockSpec((1, slab), lambda c, i: (0, c)),
            pl.BlockSpec((1, nheads), lambda c, i: (0, 0)),
            pl.BlockSpec((1, nheads), lambda c, i: (0, 0)),
            pl.BlockSpec((1, 128), lambda c, i: (0, c)),
            pl.BlockSpec((1, 128), lambda c, i: (0, c)),
        ],
        out_specs=[
            pl.BlockSpec((srows, slab // 3), lambda c, i: (i, c)),
            pl.BlockSpec((nbl, srows, nheads), lambda c, i: (c, i, 0)),
        ],
        out_shape=[
            jax.ShapeDtypeStruct((s, dcc // 3), jnp.float32),
            jax.ShapeDtypeStruct((nb, s, nheads), jnp.float32),
        ],
        scratch_shapes=[
            pltpu.VMEM((slab // 3, HP), jnp.float32),
            pltpu.VMEM((8, nheads), jnp.float32),
            pltpu.VMEM((8, slab), jnp.bfloat16),
            pltpu.VMEM((srows, slab), jnp.float32),
            pltpu.VMEM((nbl * srows, nbl * srows), jnp.bfloat16),
            pltpu.VMEM((srows + 8, slab), jnp.bfloat16),
        ],
        compiler_params=pltpu.CompilerParams(
            dimension_semantics=("core_parallel", "arbitrary")),
        name="proj_conv_ssd_norm",
    )(x, w1_bf, wdt_bf, wdtp_bf, cwp, cbp, ap_row, dtb_row,
      app_row, dtbp_row)


def _out_kernel(nheads, y_ref, inv_ref, w_ref, o_ref):
    f32 = jnp.float32
    bm = y_ref.shape[0]
    di = y_ref.shape[1]
    hp = di // nheads
    inv = inv_ref[...].reshape(bm, nheads)
    invx = jnp.concatenate(
        [jnp.broadcast_to(inv[:, h:h + 1], (bm, hp)) for h in range(nheads)],
        axis=1)
    z = (y_ref[...] * invx).astype(jnp.bfloat16)
    o = jax.lax.dot_general(
        z, w_ref[...], dimension_numbers=(((1,), (0,)), ((), ())),
        preferred_element_type=f32)
    o_ref[...] = o.reshape(1, bm, o.shape[1])


def _out_proj(y0, invn, wot_bf):
    nb, s, nheads = invn.shape
    di = y0.shape[1]
    dm = wot_bf.shape[1]
    bm = 1024
    nbl = nb // NCORES
    kfn = functools.partial(_out_kernel, nheads)
    return pl.pallas_call(
        kfn,
        grid=(NCORES, s // bm, nbl),
        in_specs=[
            pl.BlockSpec((bm, di), lambda c, m, b: (m, 0)),
            pl.BlockSpec((1, bm, nheads),
                         lambda c, m, b: (c * nbl + b, m, 0)),
            pl.BlockSpec((di, dm), lambda c, m, b: (0, 0)),
        ],
        out_specs=pl.BlockSpec((1, bm, dm),
                               lambda c, m, b: (c * nbl + b, m, 0)),
        out_shape=jax.ShapeDtypeStruct((nb, s, dm), jnp.float32),
        compiler_params=pltpu.CompilerParams(
            dimension_semantics=("core_parallel", "parallel", "parallel")),
        name="scale_outproj",
    )(y0, invn, wot_bf)


def kernel(x, W_in, conv_w, conv_b, A_param, dt_bias, W_out):
    nb, s, dm = x.shape
    nheads = A_param.shape[0]
    dcc = conv_w.shape[0]
    nh_loc = nheads // NCORES

    def permute_cols(a):
        lead = a.shape[:-1]
        ap = a.reshape(*lead, 3, NCORES, nh_loc, HP)
        ap = jnp.moveaxis(ap, -4, -3)
        return ap.reshape(*lead, dcc)


    w1t_bf = permute_cols(W_in[:dcc].T).astype(jnp.bfloat16)
    wdt = W_in[dcc:].T
    wdt_bf = wdt.astype(jnp.bfloat16)
    wdtp = jnp.zeros((dm, NCORES * 128), jnp.float32)
    app_row = jnp.zeros((1, NCORES * 128), jnp.float32)
    dtbp_row = jnp.zeros((1, NCORES * 128), jnp.float32)
    for c in range(NCORES):
        hs = slice(c * nh_loc, (c + 1) * nh_loc)
        cs = slice(c * 128, c * 128 + nh_loc)
        wdtp = wdtp.at[:, cs].set(wdt[:, hs])
        app_row = app_row.at[0, cs].set(A_param[hs])
        dtbp_row = dtbp_row.at[0, cs].set(dt_bias[hs])
    wdtp_bf = wdtp.astype(jnp.bfloat16)
    cwp = permute_cols(conv_w.T)
    cbp = permute_cols(conv_b.reshape(1, dcc))
    ap_row = A_param.reshape(1, nheads)
    dtb_row = dt_bias.reshape(1, nheads)
    wot_bf = W_out.T.astype(jnp.bfloat16)

    y0p, invn = _ssd(w1t_bf, x, wdt_bf, wdtp_bf, cwp, cbp, ap_row, dtb_row,
                     app_row, dtbp_row)
    return _out_proj(y0p, invn, wot_bf)
